# baseline (speedup 1.0000x reference)
import functools

import jax
import jax.numpy as jnp
from jax import lax
from jax.experimental import pallas as pl
from jax.experimental.pallas import tpu as pltpu
from jax.experimental.pallas import tpu_sc as plsc

CUTOFF = 12.0
CUTON = 0.8 * CUTOFF
KE = 14.399645

N_NODES = 50000
N_EDGES = 1600000
NW = 32
E_PER_W = N_EDGES // NW
B = 2000
NCH = E_PER_W // B
LANES = 16

_HI = jnp.int32(-65536)
_RND = jnp.int32(0x8000)
_F16_SCALE = 5.192296858534828e33
_F16_INV = 1.925929944387236e-34

ROWS = 12500
BR = 1250


def _sc_body(uv_hbm, ta_hbm, tb_hbm, w1_hbm, w2_hbm,
             uv_v, w1_v, w2_v, ta_v, tb_v, sem_in, sem_out, sem_tab):
    wid = lax.axis_index("s") * 2 + lax.axis_index("c")

    ta_cp = pltpu.async_copy(ta_hbm, ta_v, sem_tab)
    tb_cp = pltpu.async_copy(tb_hbm, tb_v, sem_tab)

    def out_pairs(base, bb):
        return [(w1_v.at[pl.ds(bb, B)], w1_hbm.at[pl.ds(base, B)]),
                (w2_v.at[pl.ds(bb, B)], w2_hbm.at[pl.ds(base, B)])]

    def issue_in(ci, bb):
        base = wid * E_PER_W + ci * B
        pltpu.async_copy(uv_hbm.at[pl.ds(base, B)],
                         uv_v.at[pl.ds(bb, B)], sem_in)

    issue_in(0, 0)
    ta_cp.wait()
    tb_cp.wait()

    def g_body(g, carry):
        bb = (g & 1) * B
        base = wid * E_PER_W + g * B

        @pl.when(g + 1 < NCH)
        def _prefetch():
            issue_in(g + 1, B - bb)

        pltpu.make_async_copy(uv_hbm.at[pl.ds(base, B)],
                              uv_v.at[pl.ds(bb, B)], sem_in).wait()

        @pl.when(g >= 2)
        def _guard():
            for src, dst in out_pairs(base, bb):
                pltpu.make_async_copy(src, dst, sem_out).wait()

        @plsc.parallel_loop(0, B, step=LANES, unroll=8)
        def step(s0):
            s = bb + s0
            w = uv_v[pl.ds(s, LANES)]
            iu = w & 0xFFFF
            iv = lax.shift_right_logical(w, 16)
            wa_u = plsc.load_gather(ta_v, [iu])
            wa_v = plsc.load_gather(ta_v, [iv])
            wb_u = plsc.load_gather(tb_v, [iu])
            qu = plsc.bitcast(wa_u << 16, jnp.float32)
            qv = plsc.bitcast(wa_v << 16, jnp.float32)
            dz = plsc.bitcast(wa_u & _HI, jnp.float32)
            dx = plsc.bitcast(wb_u << 16, jnp.float32)
            dy = plsc.bitcast(wb_u & _HI, jnp.float32)

            gx = plsc.bitcast(qv * dx, jnp.int32)
            gy = plsc.bitcast(qv * dy, jnp.int32)
            gz = plsc.bitcast(qv * dz, jnp.int32)
            t = plsc.bitcast(qu * qv * _F16_INV, jnp.int32)
            qq16 = ((lax.shift_right_logical(t, 16) & 0x8000)
                    | (lax.shift_right_logical(t + 0x1000, 13) & 0x7FFF))
            w1_v[pl.ds(s, LANES)] = qq16 | ((gz + _RND) & _HI)
            w2_v[pl.ds(s, LANES)] = (
                lax.shift_right_logical(gx + _RND, 16) | ((gy + _RND) & _HI))

        for src, dst in out_pairs(base, bb):
            pltpu.async_copy(src, dst, sem_out)
        return carry

    lax.fori_loop(0, NCH, g_body, 0)

    for _ in range(2):
        for ref, hbm in [(w1_v, w1_hbm), (w2_v, w2_hbm)]:
            pltpu.make_async_copy(ref.at[pl.ds(0, B)],
                                  hbm.at[pl.ds(0, B)], sem_out).wait()


def _tc_body(d_ref, vx_ref, vy_ref, vz_ref, g1_ref, g2_ref, o_ref):
    c_shift_a = 2.0 / CUTOFF
    c_shift_b = 1.0 / (CUTOFF * CUTOFF)
    inv_w = 1.0 / (CUTOFF - CUTON)

    d = d_ref[...]
    w1 = g1_ref[...]
    w2 = g2_ref[...]
    bc = lambda x: lax.bitcast_convert_type(x, jnp.float32)
    gz = bc(w1 & (-65536))
    gx = bc(w2 << 16)
    gy = bc(w2 & (-65536))
    qq = bc(((w1 & 0x8000) << 16) | ((w1 & 0x7FFF) << 13)) * _F16_SCALE

    chi = 1.0 / d
    chi_shift = c_shift_a - d * c_shift_b
    e = qq * (chi - chi_shift)
    chi2 = chi * chi
    chi2_shift = chi_shift * chi_shift
    dot = (vx_ref[...] * gx + vy_ref[...] * gy + vz_ref[...] * gz) * chi
    e = e + dot * (chi2 - chi2_shift)
    x = (d - CUTON) * inv_w
    x = jnp.minimum(jnp.maximum(x, 0.0), 1.0)
    sw = 1.0 + x * x * x * (-10.0 + x * (15.0 - 6.0 * x))
    o_ref[...] = (KE * e) * sw


def _b16(x, dtype):
    return lax.bitcast_convert_type(
        x.astype(dtype), jnp.uint16).astype(jnp.int32)


def kernel(mlmm_distances, mlmm_vectors, mlmm_atomic_charges, atomic_dipoles,
           mlmm_idxu, mlmm_idxv):
    iu = mlmm_idxu.astype(jnp.int32)
    iv = mlmm_idxv.astype(jnp.int32)
    uv = iu | (iv << 16)

    q16 = _b16(mlmm_atomic_charges, jnp.bfloat16)
    dip_t = atomic_dipoles.T
    d16 = [_b16(dip_t[i], jnp.bfloat16) for i in range(3)]
    word_a = q16 | (d16[2] << 16)
    word_b = d16[0] | (d16[1] << 16)

    mesh = plsc.VectorSubcoreMesh(core_axis_name="c", subcore_axis_name="s")
    sc_run = functools.partial(
        pl.kernel,
        out_type=(jax.ShapeDtypeStruct((N_EDGES,), jnp.int32),
                  jax.ShapeDtypeStruct((N_EDGES,), jnp.int32)),
        mesh=mesh,
        compiler_params=pltpu.CompilerParams(
            needs_layout_passes=False, use_tc_tiling_on_sc=False),
        scratch_types=[
            pltpu.VMEM((2 * B,), jnp.int32),
            pltpu.VMEM((2 * B,), jnp.int32),
            pltpu.VMEM((2 * B,), jnp.int32),
            pltpu.VMEM((N_NODES,), jnp.int32),
            pltpu.VMEM((N_NODES,), jnp.int32),
            pltpu.SemaphoreType.DMA,
            pltpu.SemaphoreType.DMA,
            pltpu.SemaphoreType.DMA,
        ],
    )(_sc_body)
    w1, w2 = sc_run(uv, word_a, word_b)

    vec_t = mlmm_vectors.T
    vx, vy, vz = vec_t[0], vec_t[1], vec_t[2]

    return pl.pallas_call(
        _tc_body,
        out_shape=jax.ShapeDtypeStruct((N_EDGES,), jnp.float32),
    )(mlmm_distances, vx, vy, vz, w1, w2)


# speedup vs baseline: 1.1116x; 1.1116x over previous
import functools

import jax
import jax.numpy as jnp
from jax import lax
from jax.experimental import pallas as pl
from jax.experimental.pallas import tpu as pltpu
from jax.experimental.pallas import tpu_sc as plsc

CUTOFF = 12.0
CUTON = 0.8 * CUTOFF
KE = 14.399645

N_NODES = 50000
N_EDGES = 1600000
NW = 32
E_PER_W = N_EDGES // NW
B = 2000
NCH = E_PER_W // B
LANES = 16

_HI = jnp.int32(-65536)
_RND = jnp.int32(0x8000)


def _sc_body(uv_hbm, ta_hbm, tb_hbm, qq_hbm, g1_hbm, g2_hbm,
             uv_v, qq_v, g1_v, g2_v, ta_v, tb_v, sem_in, sem_out):
    wid = lax.axis_index("s") * 2 + lax.axis_index("c")

    pltpu.sync_copy(ta_hbm, ta_v)
    pltpu.sync_copy(tb_hbm, tb_v)

    def out_pairs(base, bb):
        return [(qq_v.at[pl.ds(bb, B)], qq_hbm.at[pl.ds(base, B)]),
                (g1_v.at[pl.ds(bb, B)], g1_hbm.at[pl.ds(base, B)]),
                (g2_v.at[pl.ds(bb, B)], g2_hbm.at[pl.ds(base, B)])]

    def issue_in(ci, bb):
        base = wid * E_PER_W + ci * B
        pltpu.async_copy(uv_hbm.at[pl.ds(base, B)],
                         uv_v.at[pl.ds(bb, B)], sem_in)

    issue_in(0, 0)

    def g_body(g, carry):
        bb = (g & 1) * B
        base = wid * E_PER_W + g * B

        @pl.when(g + 1 < NCH)
        def _prefetch():
            issue_in(g + 1, B - bb)

        pltpu.make_async_copy(uv_hbm.at[pl.ds(base, B)],
                              uv_v.at[pl.ds(bb, B)], sem_in).wait()

        @pl.when(g >= 2)
        def _guard():
            for src, dst in out_pairs(base, bb):
                pltpu.make_async_copy(src, dst, sem_out).wait()

        @plsc.parallel_loop(0, B, step=LANES, unroll=8)
        def step(s0):
            s = bb + s0
            w = uv_v[pl.ds(s, LANES)]
            iu = w & 0xFFFF
            iv = lax.shift_right_logical(w, 16)
            wa_u = plsc.load_gather(ta_v, [iu])
            wa_v = plsc.load_gather(ta_v, [iv])
            wb_u = plsc.load_gather(tb_v, [iu])
            qu = plsc.bitcast(wa_u << 16, jnp.float32)
            qv = plsc.bitcast(wa_v << 16, jnp.float32)
            dz = plsc.bitcast(wa_u & _HI, jnp.float32)
            dx = plsc.bitcast(wb_u << 16, jnp.float32)
            dy = plsc.bitcast(wb_u & _HI, jnp.float32)

            qq_v[pl.ds(s, LANES)] = qu * qv
            gx = plsc.bitcast(qv * dx, jnp.int32)
            gy = plsc.bitcast(qv * dy, jnp.int32)
            gz = plsc.bitcast(qv * dz, jnp.int32)
            g1_v[pl.ds(s, LANES)] = (
                lax.shift_right_logical(gx + _RND, 16) | ((gy + _RND) & _HI))
            g2_v[pl.ds(s, LANES)] = lax.shift_right_logical(gz + _RND, 16)

        for src, dst in out_pairs(base, bb):
            pltpu.async_copy(src, dst, sem_out)
        return carry

    lax.fori_loop(0, NCH, g_body, 0)

    for _ in range(2):
        for ref, hbm in [(qq_v, qq_hbm), (g1_v, g1_hbm), (g2_v, g2_hbm)]:
            pltpu.make_async_copy(ref.at[pl.ds(0, B)],
                                  hbm.at[pl.ds(0, B)], sem_out).wait()


def _tc_body(d_ref, vx_ref, vy_ref, vz_ref, qq_ref, g1_ref, g2_ref, o_ref):
    c_shift_a = 2.0 / CUTOFF
    c_shift_b = 1.0 / (CUTOFF * CUTOFF)
    inv_w = 1.0 / (CUTOFF - CUTON)

    d = d_ref[...]
    g1 = g1_ref[...]
    g2 = g2_ref[...]
    bc = lambda x: lax.bitcast_convert_type(x, jnp.float32)
    gx = bc(g1 << 16)
    gy = bc(g1 & (-65536))
    gz = bc(g2 << 16)

    chi = 1.0 / d
    chi_shift = c_shift_a - d * c_shift_b
    e = qq_ref[...] * (chi - chi_shift)
    chi2 = chi * chi
    chi2_shift = chi_shift * chi_shift
    dot = (vx_ref[...] * gx + vy_ref[...] * gy + vz_ref[...] * gz) * chi
    e = e + dot * (chi2 - chi2_shift)
    x = (d - CUTON) * inv_w
    x = jnp.minimum(jnp.maximum(x, 0.0), 1.0)
    sw = 1.0 + x * x * x * (-10.0 + x * (15.0 - 6.0 * x))
    o_ref[...] = (KE * e) * sw


def _b16(x, dtype):
    return lax.bitcast_convert_type(
        x.astype(dtype), jnp.uint16).astype(jnp.int32)


def kernel(mlmm_distances, mlmm_vectors, mlmm_atomic_charges, atomic_dipoles,
           mlmm_idxu, mlmm_idxv):
    iu = mlmm_idxu.astype(jnp.int32)
    iv = mlmm_idxv.astype(jnp.int32)
    uv = iu | (iv << 16)

    q16 = _b16(mlmm_atomic_charges, jnp.bfloat16)
    dip_t = atomic_dipoles.T
    d16 = [_b16(dip_t[i], jnp.bfloat16) for i in range(3)]
    word_a = q16 | (d16[2] << 16)
    word_b = d16[0] | (d16[1] << 16)

    mesh = plsc.VectorSubcoreMesh(core_axis_name="c", subcore_axis_name="s")
    sc_run = functools.partial(
        pl.kernel,
        out_type=(jax.ShapeDtypeStruct((N_EDGES,), jnp.float32),
                  jax.ShapeDtypeStruct((N_EDGES,), jnp.int32),
                  jax.ShapeDtypeStruct((N_EDGES,), jnp.int32)),
        mesh=mesh,
        compiler_params=pltpu.CompilerParams(
            needs_layout_passes=False, use_tc_tiling_on_sc=False),
        scratch_types=[
            pltpu.VMEM((2 * B,), jnp.int32),
            pltpu.VMEM((2 * B,), jnp.float32),
            pltpu.VMEM((2 * B,), jnp.int32),
            pltpu.VMEM((2 * B,), jnp.int32),
            pltpu.VMEM((N_NODES,), jnp.int32),
            pltpu.VMEM((N_NODES,), jnp.int32),
            pltpu.SemaphoreType.DMA,
            pltpu.SemaphoreType.DMA,
        ],
    )(_sc_body)
    qq, g1, g2 = sc_run(uv, word_a, word_b)

    vec_t = mlmm_vectors.T
    vx, vy, vz = vec_t[0], vec_t[1], vec_t[2]

    return pl.pallas_call(
        _tc_body,
        out_shape=jax.ShapeDtypeStruct((N_EDGES,), jnp.float32),
    )(mlmm_distances, vx, vy, vz, qq, g1, g2)


# speedup vs baseline: 1.1127x; 1.0010x over previous
import functools

import jax
import jax.numpy as jnp
from jax import lax
from jax.experimental import pallas as pl
from jax.experimental.pallas import tpu as pltpu
from jax.experimental.pallas import tpu_sc as plsc

CUTOFF = 12.0
CUTON = 0.8 * CUTOFF
KE = 14.399645

N_NODES = 50000
N_EDGES = 1600000
NW = 32
E_PER_W = N_EDGES // NW
B = 2000
NCH = E_PER_W // B
LANES = 16

_HI = jnp.int32(-65536)
_RND = jnp.int32(0x8000)


def _sc_body(uv_hbm, ta_hbm, tb_hbm, qq_hbm, g1_hbm, g2_hbm,
             uv_v, qq_v, g1_v, g2_v, ta_v, tb_v, sem_in, sem_out):
    wid = lax.axis_index("s") * 2 + lax.axis_index("c")

    pltpu.sync_copy(ta_hbm, ta_v)
    pltpu.sync_copy(tb_hbm, tb_v)

    def out_pairs(base, bb):
        return [(qq_v.at[pl.ds(bb, B)], qq_hbm.at[pl.ds(base, B)]),
                (g1_v.at[pl.ds(bb, B)], g1_hbm.at[pl.ds(base, B)]),
                (g2_v.at[pl.ds(bb, B)], g2_hbm.at[pl.ds(base, B)])]

    def issue_in(ci, bb):
        base = wid * E_PER_W + ci * B
        pltpu.async_copy(uv_hbm.at[pl.ds(base, B)],
                         uv_v.at[pl.ds(bb, B)], sem_in)

    issue_in(0, 0)

    def g_body(g, carry):
        bb = (g & 1) * B
        base = wid * E_PER_W + g * B

        @pl.when(g + 1 < NCH)
        def _prefetch():
            issue_in(g + 1, B - bb)

        pltpu.make_async_copy(uv_hbm.at[pl.ds(base, B)],
                              uv_v.at[pl.ds(bb, B)], sem_in).wait()

        @pl.when(g >= 2)
        def _guard():
            for src, dst in out_pairs(base, bb):
                pltpu.make_async_copy(src, dst, sem_out).wait()

        @plsc.parallel_loop(0, B, step=LANES, unroll=16)
        def step(s0):
            s = bb + s0
            w = uv_v[pl.ds(s, LANES)]
            iu = w & 0xFFFF
            iv = lax.shift_right_logical(w, 16)
            wa_u = plsc.load_gather(ta_v, [iu])
            wa_v = plsc.load_gather(ta_v, [iv])
            wb_u = plsc.load_gather(tb_v, [iu])
            qu = plsc.bitcast(wa_u << 16, jnp.float32)
            qv = plsc.bitcast(wa_v << 16, jnp.float32)
            dz = plsc.bitcast(wa_u & _HI, jnp.float32)
            dx = plsc.bitcast(wb_u << 16, jnp.float32)
            dy = plsc.bitcast(wb_u & _HI, jnp.float32)

            qq_v[pl.ds(s, LANES)] = qu * qv
            gx = plsc.bitcast(qv * dx, jnp.int32)
            gy = plsc.bitcast(qv * dy, jnp.int32)
            gz = plsc.bitcast(qv * dz, jnp.int32)
            g1_v[pl.ds(s, LANES)] = (
                lax.shift_right_logical(gx + _RND, 16) | ((gy + _RND) & _HI))
            g2_v[pl.ds(s, LANES)] = lax.shift_right_logical(gz + _RND, 16)

        for src, dst in out_pairs(base, bb):
            pltpu.async_copy(src, dst, sem_out)
        return carry

    lax.fori_loop(0, NCH, g_body, 0)

    for _ in range(2):
        for ref, hbm in [(qq_v, qq_hbm), (g1_v, g1_hbm), (g2_v, g2_hbm)]:
            pltpu.make_async_copy(ref.at[pl.ds(0, B)],
                                  hbm.at[pl.ds(0, B)], sem_out).wait()


def _tc_body(d_ref, vx_ref, vy_ref, vz_ref, qq_ref, g1_ref, g2_ref, o_ref):
    c_shift_a = 2.0 / CUTOFF
    c_shift_b = 1.0 / (CUTOFF * CUTOFF)
    inv_w = 1.0 / (CUTOFF - CUTON)

    d = d_ref[...]
    g1 = g1_ref[...]
    g2 = g2_ref[...]
    bc = lambda x: lax.bitcast_convert_type(x, jnp.float32)
    gx = bc(g1 << 16)
    gy = bc(g1 & (-65536))
    gz = bc(g2 << 16)

    chi = 1.0 / d
    chi_shift = c_shift_a - d * c_shift_b
    e = qq_ref[...] * (chi - chi_shift)
    chi2 = chi * chi
    chi2_shift = chi_shift * chi_shift
    dot = (vx_ref[...] * gx + vy_ref[...] * gy + vz_ref[...] * gz) * chi
    e = e + dot * (chi2 - chi2_shift)
    x = (d - CUTON) * inv_w
    x = jnp.minimum(jnp.maximum(x, 0.0), 1.0)
    sw = 1.0 + x * x * x * (-10.0 + x * (15.0 - 6.0 * x))
    o_ref[...] = (KE * e) * sw


def _b16(x, dtype):
    return lax.bitcast_convert_type(
        x.astype(dtype), jnp.uint16).astype(jnp.int32)


def kernel(mlmm_distances, mlmm_vectors, mlmm_atomic_charges, atomic_dipoles,
           mlmm_idxu, mlmm_idxv):
    iu = mlmm_idxu.astype(jnp.int32)
    iv = mlmm_idxv.astype(jnp.int32)
    uv = iu | (iv << 16)

    q16 = _b16(mlmm_atomic_charges, jnp.bfloat16)
    dip_t = atomic_dipoles.T
    d16 = [_b16(dip_t[i], jnp.bfloat16) for i in range(3)]
    word_a = q16 | (d16[2] << 16)
    word_b = d16[0] | (d16[1] << 16)

    mesh = plsc.VectorSubcoreMesh(core_axis_name="c", subcore_axis_name="s")
    sc_run = functools.partial(
        pl.kernel,
        out_type=(jax.ShapeDtypeStruct((N_EDGES,), jnp.float32),
                  jax.ShapeDtypeStruct((N_EDGES,), jnp.int32),
                  jax.ShapeDtypeStruct((N_EDGES,), jnp.int32)),
        mesh=mesh,
        compiler_params=pltpu.CompilerParams(
            needs_layout_passes=False, use_tc_tiling_on_sc=False),
        scratch_types=[
            pltpu.VMEM((2 * B,), jnp.int32),
            pltpu.VMEM((2 * B,), jnp.float32),
            pltpu.VMEM((2 * B,), jnp.int32),
            pltpu.VMEM((2 * B,), jnp.int32),
            pltpu.VMEM((N_NODES,), jnp.int32),
            pltpu.VMEM((N_NODES,), jnp.int32),
            pltpu.SemaphoreType.DMA,
            pltpu.SemaphoreType.DMA,
        ],
    )(_sc_body)
    qq, g1, g2 = sc_run(uv, word_a, word_b)

    vec_t = mlmm_vectors.T
    vx, vy, vz = vec_t[0], vec_t[1], vec_t[2]

    return pl.pallas_call(
        _tc_body,
        out_shape=jax.ShapeDtypeStruct((N_EDGES,), jnp.float32),
    )(mlmm_distances, vx, vy, vz, qq, g1, g2)
